# baseline (speedup 1.0000x reference)
import jax
import jax.numpy as jnp
from jax.experimental import pallas as pl
from jax.experimental.pallas import tpu as pltpu

NUM_ADAPTERS = 8
IN_FEATURES = 4096
OUT_FEATURES = 4096
BATCH = 4
SEQ_LEN = 2048

BS = 2048
BO = 256


def _lora_mm_kernel(ids_ref, x_ref, w_ref, o_ref):
    x = x_ref[0]
    w = w_ref[0].astype(jnp.bfloat16)
    o_ref[0] = jax.lax.dot_general(
        x, w, (((1,), (1,)), ((), ())),
        preferred_element_type=jnp.float32)


def kernel(x, adapter_ids, weight):
    x = x.astype(jnp.bfloat16)
    grid = (BATCH, OUT_FEATURES // BO)
    return pl.pallas_call(
        _lora_mm_kernel,
        grid_spec=pltpu.PrefetchScalarGridSpec(
            num_scalar_prefetch=1,
            grid=grid,
            in_specs=[
                pl.BlockSpec((1, BS, IN_FEATURES),
                             lambda b, o, ids: (b, 0, 0)),
                pl.BlockSpec((1, BO, IN_FEATURES),
                             lambda b, o, ids: (ids[b], o, 0)),
            ],
            out_specs=pl.BlockSpec((1, BS, BO),
                                   lambda b, o, ids: (b, 0, o)),
        ),
        out_shape=jax.ShapeDtypeStruct((BATCH, SEQ_LEN, OUT_FEATURES),
                                       jnp.float32),
        compiler_params=pltpu.CompilerParams(
            dimension_semantics=("parallel", "arbitrary")),
    )(adapter_ids, x, weight)


# speedup vs baseline: 1.1331x; 1.1331x over previous
import jax
import jax.numpy as jnp
from jax.experimental import pallas as pl
from jax.experimental.pallas import tpu as pltpu

NUM_ADAPTERS = 8
IN_FEATURES = 4096
OUT_FEATURES = 4096
BATCH = 4
SEQ_LEN = 2048

BS = 1024
BO = 512


def _lora_mm_kernel(ids_ref, x_ref, w_ref, o_ref):
    x = x_ref[0]
    w = w_ref[0]
    o_ref[0] = jax.lax.dot_general(
        x, w, (((1,), (1,)), ((), ())),
        preferred_element_type=jnp.float32)


def kernel(x, adapter_ids, weight):
    grid = (BATCH, SEQ_LEN // BS, OUT_FEATURES // BO)
    return pl.pallas_call(
        _lora_mm_kernel,
        grid_spec=pltpu.PrefetchScalarGridSpec(
            num_scalar_prefetch=1,
            grid=grid,
            in_specs=[
                pl.BlockSpec((1, BS, IN_FEATURES),
                             lambda b, s, o, ids: (b, s, 0)),
                pl.BlockSpec((1, BO, IN_FEATURES),
                             lambda b, s, o, ids: (ids[b], o, 0)),
            ],
            out_specs=pl.BlockSpec((1, BS, BO),
                                   lambda b, s, o, ids: (b, s, o)),
        ),
        out_shape=jax.ShapeDtypeStruct((BATCH, SEQ_LEN, OUT_FEATURES),
                                       jnp.float32),
        compiler_params=pltpu.CompilerParams(
            dimension_semantics=("parallel", "parallel", "arbitrary")),
    )(adapter_ids, x, weight)


# speedup vs baseline: 1.1339x; 1.0007x over previous
import jax
import jax.numpy as jnp
from jax.experimental import pallas as pl
from jax.experimental.pallas import tpu as pltpu

NUM_ADAPTERS = 8
IN_FEATURES = 4096
OUT_FEATURES = 4096
BATCH = 4
SEQ_LEN = 2048

BS = 1024
BO = 512


def _lora_mm_kernel(ids_ref, x_ref, w_ref, o_ref):
    x = x_ref[0].astype(jnp.bfloat16)
    w = w_ref[0].astype(jnp.bfloat16)
    o_ref[0] = jax.lax.dot_general(
        x, w, (((1,), (1,)), ((), ())),
        preferred_element_type=jnp.float32)


def kernel(x, adapter_ids, weight):
    grid = (BATCH, SEQ_LEN // BS, OUT_FEATURES // BO)
    return pl.pallas_call(
        _lora_mm_kernel,
        grid_spec=pltpu.PrefetchScalarGridSpec(
            num_scalar_prefetch=1,
            grid=grid,
            in_specs=[
                pl.BlockSpec((1, BS, IN_FEATURES),
                             lambda b, s, o, ids: (b, s, 0)),
                pl.BlockSpec((1, BO, IN_FEATURES),
                             lambda b, s, o, ids: (ids[b], o, 0)),
            ],
            out_specs=pl.BlockSpec((1, BS, BO),
                                   lambda b, s, o, ids: (b, s, o)),
        ),
        out_shape=jax.ShapeDtypeStruct((BATCH, SEQ_LEN, OUT_FEATURES),
                                       jnp.float32),
        compiler_params=pltpu.CompilerParams(
            dimension_semantics=("parallel", "parallel", "arbitrary")),
    )(adapter_ids, x, weight)
